# baseline (speedup 1.0000x reference)
import functools

import jax
import jax.numpy as jnp
from jax import lax
from jax.experimental import pallas as pl
from jax.experimental.pallas import tpu as pltpu
from jax.experimental.pallas import tpu_sc as plsc

NC = 2
NS = 16
NW = NC * NS
LANES = 16
CHUNK = 40
CH = 128

_F32 = jnp.float32
_HIGH = jax.lax.Precision.HIGHEST


def _vector_mesh():
    return plsc.VectorSubcoreMesh(core_axis_name="c", subcore_axis_name="s")


def _sc_params():
    import dataclasses
    cp = pltpu.CompilerParams()
    return dataclasses.replace(cp, needs_layout_passes=False)


def _deg_body(dst_hbm, w_hbm, out_hbm, acc, dstv, wv):
    n = acc.shape[1]
    ept = dst_hbm.shape[0] // NW
    cid = lax.axis_index("c")
    sid = lax.axis_index("s")
    wid = sid * NC + cid
    zeros16 = jnp.zeros((LANES,), _F32)
    zeros16i = jnp.zeros((LANES,), jnp.int32)

    @pl.loop(0, n, step=LANES)
    def _(j):
        acc[0, pl.ds(j, LANES)] = zeros16

    pltpu.sync_copy(dst_hbm.at[pl.ds(wid * ept, ept)], dstv)
    pltpu.sync_copy(w_hbm.at[pl.ds(wid * ept, ept)], wv)

    @pl.loop(0, ept, step=LANES)
    def _(j):
        plsc.addupdate_scatter(acc, [zeros16i, dstv[pl.ds(j, LANES)]],
                               wv[pl.ds(j, LANES)])

    pltpu.sync_copy(acc, out_hbm.at[wid])


def _deg_partials(dst, w, n):
    ept = dst.shape[0] // NW
    kern = functools.partial(
        pl.kernel,
        out_type=jax.ShapeDtypeStruct((NW, 1, n), _F32),
        mesh=_vector_mesh(),
        compiler_params=_sc_params(),
        scratch_types=[
            pltpu.VMEM((1, n), _F32),
            pltpu.VMEM((ept,), jnp.int32),
            pltpu.VMEM((ept,), _F32),
        ],
    )(_deg_body)
    return kern(dst, w)


def _agg_body(y_hbm, src_hbm, dst_hbm, w_hbm, out_hbm,
              acc_sh, src_v, dst_v, w_v, rows_a, rows_b,
              sem_a, sem_b):
    d = y_hbm.shape[1]
    npad = acc_sh.shape[0]
    nch = src_hbm.shape[1]
    nph = nch // src_v.shape[0]
    hch = src_v.shape[0]
    rpt = npad // NS
    zrows = rows_a.shape[0]
    cid = lax.axis_index("c")
    sid = lax.axis_index("s")
    wid = sid * NC + cid
    zeros16 = jnp.zeros((LANES,), _F32)

    @pl.loop(0, zrows)
    def _(r):
        for cc in range(0, d, LANES):
            rows_a[r, pl.ds(cc, LANES)] = zeros16

    @pl.loop(0, rpt // zrows)
    def _(k):
        pltpu.sync_copy(rows_a,
                        acc_sh.at[pl.ds(sid * rpt + k * zrows, zrows)])

    plsc.subcore_barrier()

    half = CH // 2

    def start_gather(j, buf, sem):
        pltpu.make_async_copy(y_hbm.at[src_v.at[j, pl.ds(0, half)]],
                              buf.at[pl.ds(0, half)], sem).start()
        pltpu.make_async_copy(y_hbm.at[src_v.at[j, pl.ds(half, half)]],
                              buf.at[pl.ds(half, half)], sem).start()

    def wait_gather(j, buf, sem):
        pltpu.make_async_copy(y_hbm.at[src_v.at[j, pl.ds(0, half)]],
                              buf.at[pl.ds(0, half)], sem).wait()
        pltpu.make_async_copy(y_hbm.at[src_v.at[j, pl.ds(half, half)]],
                              buf.at[pl.ds(half, half)], sem).wait()

    def process(j, buf):
        wj = jnp.full((LANES,), j, jnp.int32)

        @pl.loop(0, CH, step=4)
        def _(r):
            for u in range(4):
                wsplat = plsc.load_gather(
                    w_v, [wj, jnp.full((LANES,), r + u, jnp.int32)])
                for cc in range(0, d, LANES):
                    buf[r + u, pl.ds(cc, LANES)] = (
                        buf[r + u, pl.ds(cc, LANES)] * wsplat)

        pltpu.sync_copy(buf, acc_sh.at[dst_v.at[j]], add=True)

    @pl.loop(0, nph)
    def _(ph):
        pltpu.sync_copy(src_hbm.at[wid, pl.ds(ph * hch, hch)], src_v)
        pltpu.sync_copy(dst_hbm.at[wid, pl.ds(ph * hch, hch)], dst_v)
        pltpu.sync_copy(w_hbm.at[wid, pl.ds(ph * hch, hch)], w_v)

        start_gather(0, rows_a, sem_a)

        @pl.loop(0, hch, step=2)
        def _(i):
            wait_gather(i, rows_a, sem_a)
            start_gather(i + 1, rows_b, sem_b)
            process(i, rows_a)
            wait_gather(i + 1, rows_b, sem_b)

            @pl.when(i + 2 < hch)
            def _():
                start_gather(i + 2, rows_a, sem_a)

            process(i + 1, rows_b)

    plsc.subcore_barrier()

    @pl.loop(0, rpt // zrows)
    def _(k):
        r0 = sid * rpt + k * zrows
        pltpu.sync_copy(acc_sh.at[pl.ds(r0, zrows)], rows_a)
        pltpu.sync_copy(rows_a, out_hbm.at[cid, pl.ds(r0, zrows)])


def _edge_aggregate(y, src, dst, w):
    n, d = y.shape
    e = src.shape[0]
    nch = -(-e // (NW * CH))
    if nch % 2:
        nch += 1
    epad = NW * CH * nch - e
    npad = NS * 640
    assert n <= npad
    if epad:
        src = jnp.concatenate([src, jnp.zeros((epad,), src.dtype)])
        pad_dst = n + jnp.arange(epad, dtype=dst.dtype) % (npad - n)
        dst = jnp.concatenate([dst, pad_dst])
        w = jnp.concatenate([w, jnp.zeros((epad,), w.dtype)])
    kern = functools.partial(
        pl.kernel,
        out_type=jax.ShapeDtypeStruct((NC, npad, d), _F32),
        mesh=_vector_mesh(),
        compiler_params=_sc_params(),
        scratch_types=[
            pltpu.VMEM_SHARED((npad, d), _F32),
            pltpu.VMEM((nch // 2, CH), jnp.int32),
            pltpu.VMEM((nch // 2, CH), jnp.int32),
            pltpu.VMEM((nch // 2, CH), _F32),
            pltpu.VMEM((CH, d), _F32),
            pltpu.VMEM((CH, d), _F32),
            pltpu.SemaphoreType.DMA,
            pltpu.SemaphoreType.DMA,
        ],
    )(_agg_body)
    return kern(y, src.reshape(NW, nch, CH), dst.reshape(NW, nch, CH),
                w.reshape(NW, nch, CH))[:, :n]


def _leaky(v):
    return jnp.where(v >= 0, v, 0.01 * v)


def _k1_body(x_ref, w_ref, dinv_ref, y_ref):
    y_ref[...] = jnp.dot(x_ref[...], w_ref[...], precision=_HIGH,
                         preferred_element_type=_F32) * dinv_ref[...]


def _first_matmul(x, w1, dinv):
    return pl.pallas_call(
        _k1_body,
        out_shape=jax.ShapeDtypeStruct(x.shape, _F32),
    )(x, w1, dinv)


def _kmid_body(y_ref, parts_ref, dinv_ref, b_ref, w_ref, o_ref):
    h = y_ref[...] + parts_ref[0] + parts_ref[1]
    h = _leaky(dinv_ref[...] * h + b_ref[...])
    o_ref[...] = jnp.dot(h, w_ref[...], precision=_HIGH,
                         preferred_element_type=_F32) * dinv_ref[...]


def _mid_layer(y, parts, dinv, b, w_next):
    return pl.pallas_call(
        _kmid_body,
        out_shape=jax.ShapeDtypeStruct(y.shape, _F32),
    )(y, parts, dinv, b, w_next)


def _kfin_body(y_ref, parts_ref, dinv_ref, b_ref, batch_ref, wfc_ref,
               bfc_ref, o_ref):
    n = y_ref.shape[0]
    g = o_ref.shape[0]
    h = y_ref[...] + parts_ref[0] + parts_ref[1]
    h = dinv_ref[...] * h + b_ref[...]
    seg = lax.broadcasted_iota(jnp.int32, (n, g), 1)
    oh = (batch_ref[...] == seg).astype(_F32)
    sums = lax.dot_general(oh, h, (((0,), (0,)), ((), ())),
                           precision=_HIGH, preferred_element_type=_F32)
    counts = lax.dot_general(oh, jnp.ones((n, 1), _F32),
                             (((0,), (0,)), ((), ())),
                             precision=_HIGH, preferred_element_type=_F32)
    pooled = sums / jnp.maximum(counts, 1.0)
    logits = jnp.dot(pooled, wfc_ref[...], precision=_HIGH,
                     preferred_element_type=_F32) + bfc_ref[...]
    o_ref[...] = jax.nn.sigmoid(logits)


def _final_stage(y3, parts, dinv, b3, batch, wfc, bfc, g):
    o = wfc.shape[1]
    return pl.pallas_call(
        _kfin_body,
        out_shape=jax.ShapeDtypeStruct((g, o), _F32),
    )(y3, parts, dinv, b3, batch, wfc, bfc)


def kernel(x, edge_index, edge_weight, batch, W1, b1, W2, b2, W3, b3,
           Wfc, bfc):
    n, d = x.shape
    src = edge_index[0]
    dst = edge_index[1]

    degp = _deg_partials(dst, edge_weight, n)
    deg = jnp.sum(degp.reshape(NW, n), axis=0) + 1.0
    dinv = jnp.where(deg > 0, lax.rsqrt(deg), 0.0).reshape(n, 1)

    y1 = _first_matmul(x, W1, dinv)
    p1 = _edge_aggregate(y1, src, dst, edge_weight)
    y2 = _mid_layer(y1, p1, dinv, b1.reshape(1, d), W2)
    p2 = _edge_aggregate(y2, src, dst, edge_weight)
    y3 = _mid_layer(y2, p2, dinv, b2.reshape(1, d), W3)
    p3 = _edge_aggregate(y3, src, dst, edge_weight)

    return _final_stage(y3, p3, dinv, b3.reshape(1, d),
                        batch.reshape(n, 1), Wfc, bfc, g=64)


# speedup vs baseline: 1.3876x; 1.3876x over previous
import functools

import jax
import jax.numpy as jnp
from jax import lax
from jax.experimental import pallas as pl
from jax.experimental.pallas import tpu as pltpu
from jax.experimental.pallas import tpu_sc as plsc

NC = 2
NS = 16
NW = NC * NS
LANES = 16
CHUNK = 40
CH = 128
CPH0 = 3
CPH1 = 1

_F32 = jnp.float32
_HIGH = jax.lax.Precision.HIGHEST


def _vector_mesh():
    return plsc.VectorSubcoreMesh(core_axis_name="c", subcore_axis_name="s")


def _sc_params():
    import dataclasses
    cp = pltpu.CompilerParams()
    return dataclasses.replace(cp, needs_layout_passes=False)


def _deg_body(dst_hbm, w_hbm, out_hbm, acc, dstv, wv):
    n = acc.shape[1]
    ept = dst_hbm.shape[0] // NW
    cid = lax.axis_index("c")
    sid = lax.axis_index("s")
    wid = sid * NC + cid
    zeros16 = jnp.zeros((LANES,), _F32)
    zeros16i = jnp.zeros((LANES,), jnp.int32)

    @pl.loop(0, n, step=LANES)
    def _(j):
        acc[0, pl.ds(j, LANES)] = zeros16

    pltpu.sync_copy(dst_hbm.at[pl.ds(wid * ept, ept)], dstv)
    pltpu.sync_copy(w_hbm.at[pl.ds(wid * ept, ept)], wv)

    @pl.loop(0, ept, step=LANES)
    def _(j):
        plsc.addupdate_scatter(acc, [zeros16i, dstv[pl.ds(j, LANES)]],
                               wv[pl.ds(j, LANES)])

    pltpu.sync_copy(acc, out_hbm.at[wid])


def _deg_partials(dst, w, n):
    ept = dst.shape[0] // NW
    kern = functools.partial(
        pl.kernel,
        out_type=jax.ShapeDtypeStruct((NW, 1, n), _F32),
        mesh=_vector_mesh(),
        compiler_params=_sc_params(),
        scratch_types=[
            pltpu.VMEM((1, n), _F32),
            pltpu.VMEM((ept,), jnp.int32),
            pltpu.VMEM((ept,), _F32),
        ],
    )(_deg_body)
    return kern(dst, w)


def _agg_body(y_hbm, src_hbm, dst_hbm, w_hbm, out_hbm,
              acc_sh, src_v, dst_v, w_v, rows_a, rows_b,
              sem_a, sem_b):
    d = y_hbm.shape[1]
    npad = acc_sh.shape[0]
    hch = src_v.shape[0]
    rpt = npad // NS
    zrows = rows_a.shape[0]
    cid = lax.axis_index("c")
    sid = lax.axis_index("s")
    zeros16 = jnp.zeros((LANES,), _F32)

    nph = jnp.where(cid == 0, CPH0, CPH1)
    cb = jnp.where(cid == 0, sid * (CPH0 * hch),
                   NS * CPH0 * hch + sid * (CPH1 * hch))

    @pl.loop(0, zrows)
    def _(r):
        for cc in range(0, d, LANES):
            rows_a[r, pl.ds(cc, LANES)] = zeros16

    @pl.loop(0, rpt // zrows)
    def _(k):
        pltpu.sync_copy(rows_a,
                        acc_sh.at[pl.ds(sid * rpt + k * zrows, zrows)])

    plsc.subcore_barrier()

    def start_gather(j, buf, sem):
        pltpu.make_async_copy(y_hbm.at[src_v.at[j]], buf, sem).start()

    def wait_gather(j, buf, sem):
        pltpu.make_async_copy(y_hbm.at[src_v.at[j]], buf, sem).wait()

    def process(j, buf):
        wj = jnp.full((LANES,), j, jnp.int32)

        @pl.loop(0, CH, step=4)
        def _(r):
            for u in range(4):
                wsplat = plsc.load_gather(
                    w_v, [wj, jnp.full((LANES,), r + u, jnp.int32)])
                for cc in range(0, d, LANES):
                    buf[r + u, pl.ds(cc, LANES)] = (
                        buf[r + u, pl.ds(cc, LANES)] * wsplat)

        pltpu.sync_copy(buf, acc_sh.at[dst_v.at[j]], add=True)

    @pl.loop(0, nph)
    def _(ph):
        c0 = cb + ph * hch
        pltpu.sync_copy(src_hbm.at[pl.ds(c0, hch)], src_v)
        pltpu.sync_copy(dst_hbm.at[pl.ds(c0, hch)], dst_v)
        pltpu.sync_copy(w_hbm.at[pl.ds(c0, hch)], w_v)

        start_gather(0, rows_a, sem_a)

        @pl.loop(0, hch, step=2)
        def _(i):
            wait_gather(i, rows_a, sem_a)
            start_gather(i + 1, rows_b, sem_b)
            process(i, rows_a)
            wait_gather(i + 1, rows_b, sem_b)

            @pl.when(i + 2 < hch)
            def _():
                start_gather(i + 2, rows_a, sem_a)

            process(i + 1, rows_b)

    plsc.subcore_barrier()

    @pl.loop(0, rpt // zrows)
    def _(k):
        r0 = sid * rpt + k * zrows
        pltpu.sync_copy(acc_sh.at[pl.ds(r0, zrows)], rows_a)
        pltpu.sync_copy(rows_a, out_hbm.at[cid, pl.ds(r0, zrows)])


def _edge_aggregate(y, src, dst, w):
    n, d = y.shape
    e = src.shape[0]
    hch = 40
    totch = NS * (CPH0 + CPH1) * hch
    epad = totch * CH - e
    assert epad >= 0
    npad = NS * 640
    assert n <= npad
    if epad:
        src = jnp.concatenate([src, jnp.zeros((epad,), src.dtype)])
        pad_dst = n + jnp.arange(epad, dtype=dst.dtype) % (npad - n)
        dst = jnp.concatenate([dst, pad_dst])
        w = jnp.concatenate([w, jnp.zeros((epad,), w.dtype)])
    kern = functools.partial(
        pl.kernel,
        out_type=jax.ShapeDtypeStruct((NC, npad, d), _F32),
        mesh=_vector_mesh(),
        compiler_params=_sc_params(),
        scratch_types=[
            pltpu.VMEM_SHARED((npad, d), _F32),
            pltpu.VMEM((hch, CH), jnp.int32),
            pltpu.VMEM((hch, CH), jnp.int32),
            pltpu.VMEM((hch, CH), _F32),
            pltpu.VMEM((CH, d), _F32),
            pltpu.VMEM((CH, d), _F32),
            pltpu.SemaphoreType.DMA,
            pltpu.SemaphoreType.DMA,
        ],
    )(_agg_body)
    return kern(y, src.reshape(totch, CH), dst.reshape(totch, CH),
                w.reshape(totch, CH))[:, :n]


def _leaky(v):
    return jnp.where(v >= 0, v, 0.01 * v)


def _k1_body(x_ref, w_ref, dinv_ref, y_ref):
    y_ref[...] = jnp.dot(x_ref[...], w_ref[...], precision=_HIGH,
                         preferred_element_type=_F32) * dinv_ref[...]


def _first_matmul(x, w1, dinv):
    return pl.pallas_call(
        _k1_body,
        out_shape=jax.ShapeDtypeStruct(x.shape, _F32),
    )(x, w1, dinv)


def _kmid_body(y_ref, parts_ref, dinv_ref, b_ref, w_ref, o_ref):
    h = y_ref[...] + parts_ref[0] + parts_ref[1]
    h = _leaky(dinv_ref[...] * h + b_ref[...])
    o_ref[...] = jnp.dot(h, w_ref[...], precision=_HIGH,
                         preferred_element_type=_F32) * dinv_ref[...]


def _mid_layer(y, parts, dinv, b, w_next):
    return pl.pallas_call(
        _kmid_body,
        out_shape=jax.ShapeDtypeStruct(y.shape, _F32),
    )(y, parts, dinv, b, w_next)


def _kfin_body(y_ref, parts_ref, dinv_ref, b_ref, batch_ref, wfc_ref,
               bfc_ref, o_ref):
    n = y_ref.shape[0]
    g = o_ref.shape[0]
    h = y_ref[...] + parts_ref[0] + parts_ref[1]
    h = dinv_ref[...] * h + b_ref[...]
    seg = lax.broadcasted_iota(jnp.int32, (n, g), 1)
    oh = (batch_ref[...] == seg).astype(_F32)
    sums = lax.dot_general(oh, h, (((0,), (0,)), ((), ())),
                           precision=_HIGH, preferred_element_type=_F32)
    counts = lax.dot_general(oh, jnp.ones((n, 1), _F32),
                             (((0,), (0,)), ((), ())),
                             precision=_HIGH, preferred_element_type=_F32)
    pooled = sums / jnp.maximum(counts, 1.0)
    logits = jnp.dot(pooled, wfc_ref[...], precision=_HIGH,
                     preferred_element_type=_F32) + bfc_ref[...]
    o_ref[...] = jax.nn.sigmoid(logits)


def _final_stage(y3, parts, dinv, b3, batch, wfc, bfc, g):
    o = wfc.shape[1]
    return pl.pallas_call(
        _kfin_body,
        out_shape=jax.ShapeDtypeStruct((g, o), _F32),
    )(y3, parts, dinv, b3, batch, wfc, bfc)


def kernel(x, edge_index, edge_weight, batch, W1, b1, W2, b2, W3, b3,
           Wfc, bfc):
    n, d = x.shape
    src = edge_index[0]
    dst = edge_index[1]

    degp = _deg_partials(dst, edge_weight, n)
    deg = jnp.sum(degp.reshape(NW, n), axis=0) + 1.0
    dinv = jnp.where(deg > 0, lax.rsqrt(deg), 0.0).reshape(n, 1)

    y1 = _first_matmul(x, W1, dinv)
    p1 = _edge_aggregate(y1, src, dst, edge_weight)
    y2 = _mid_layer(y1, p1, dinv, b1.reshape(1, d), W2)
    p2 = _edge_aggregate(y2, src, dst, edge_weight)
    y3 = _mid_layer(y2, p2, dinv, b2.reshape(1, d), W3)
    p3 = _edge_aggregate(y3, src, dst, edge_weight)

    return _final_stage(y3, p3, dinv, b3.reshape(1, d),
                        batch.reshape(n, 1), Wfc, bfc, g=64)


# speedup vs baseline: 1.4771x; 1.0645x over previous
import functools

import jax
import jax.numpy as jnp
from jax import lax
from jax.experimental import pallas as pl
from jax.experimental.pallas import tpu as pltpu
from jax.experimental.pallas import tpu_sc as plsc

NC = 2
NS = 16
NW = NC * NS
LANES = 16
CHUNK = 40
CH = 128
CPH0 = 9
CPH1 = 1
HCH = 16

_F32 = jnp.float32
_HIGH = jax.lax.Precision.HIGHEST


def _vector_mesh():
    return plsc.VectorSubcoreMesh(core_axis_name="c", subcore_axis_name="s")


def _sc_params():
    import dataclasses
    cp = pltpu.CompilerParams()
    return dataclasses.replace(cp, needs_layout_passes=False)


def _deg_body(dst_hbm, w_hbm, out_hbm, acc, dstv, wv):
    n = acc.shape[1]
    ept = dst_hbm.shape[0] // NW
    cid = lax.axis_index("c")
    sid = lax.axis_index("s")
    wid = sid * NC + cid
    zeros16 = jnp.zeros((LANES,), _F32)
    zeros16i = jnp.zeros((LANES,), jnp.int32)

    @pl.loop(0, n, step=LANES)
    def _(j):
        acc[0, pl.ds(j, LANES)] = zeros16

    pltpu.sync_copy(dst_hbm.at[pl.ds(wid * ept, ept)], dstv)
    pltpu.sync_copy(w_hbm.at[pl.ds(wid * ept, ept)], wv)

    @pl.loop(0, ept, step=LANES)
    def _(j):
        plsc.addupdate_scatter(acc, [zeros16i, dstv[pl.ds(j, LANES)]],
                               wv[pl.ds(j, LANES)])

    pltpu.sync_copy(acc, out_hbm.at[wid])


def _deg_partials(dst, w, n):
    ept = dst.shape[0] // NW
    kern = functools.partial(
        pl.kernel,
        out_type=jax.ShapeDtypeStruct((NW, 1, n), _F32),
        mesh=_vector_mesh(),
        compiler_params=_sc_params(),
        scratch_types=[
            pltpu.VMEM((1, n), _F32),
            pltpu.VMEM((ept,), jnp.int32),
            pltpu.VMEM((ept,), _F32),
        ],
    )(_deg_body)
    return kern(dst, w)


def _agg_body(y_hbm, src_hbm, dst_hbm, w_hbm, out_hbm,
              acc_sh, src_v, dst_v, w_v, rows_a, rows_b,
              sem_a, sem_b):
    d = y_hbm.shape[1]
    npad = acc_sh.shape[0]
    hch = src_v.shape[0]
    rpt = npad // NS
    zrows = rows_a.shape[0]
    cid = lax.axis_index("c")
    sid = lax.axis_index("s")
    zeros16 = jnp.zeros((LANES,), _F32)

    nph = jnp.where(cid == 0, CPH0, CPH1)
    cb = jnp.where(cid == 0, sid * (CPH0 * hch),
                   NS * CPH0 * hch + sid * (CPH1 * hch))

    @pl.loop(0, zrows)
    def _(r):
        for cc in range(0, d, LANES):
            rows_a[r, pl.ds(cc, LANES)] = zeros16

    @pl.loop(0, rpt // zrows)
    def _(k):
        pltpu.sync_copy(rows_a,
                        acc_sh.at[pl.ds(sid * rpt + k * zrows, zrows)])

    plsc.subcore_barrier()

    def start_gather(j, buf, sem):
        pltpu.make_async_copy(y_hbm.at[src_v.at[j]], buf, sem).start()

    def wait_gather(j, buf, sem):
        pltpu.make_async_copy(y_hbm.at[src_v.at[j]], buf, sem).wait()

    def process(j, buf):
        wj = jnp.full((LANES,), j, jnp.int32)

        @pl.loop(0, CH, step=4)
        def _(r):
            for u in range(4):
                wsplat = plsc.load_gather(
                    w_v, [wj, jnp.full((LANES,), r + u, jnp.int32)])
                for cc in range(0, d, LANES):
                    buf[r + u, pl.ds(cc, LANES)] = (
                        buf[r + u, pl.ds(cc, LANES)] * wsplat)

        pltpu.sync_copy(buf, acc_sh.at[dst_v.at[j]], add=True)

    @pl.loop(0, nph)
    def _(ph):
        c0 = cb + ph * hch
        pltpu.sync_copy(src_hbm.at[pl.ds(c0, hch)], src_v)
        pltpu.sync_copy(dst_hbm.at[pl.ds(c0, hch)], dst_v)
        pltpu.sync_copy(w_hbm.at[pl.ds(c0, hch)], w_v)

        start_gather(0, rows_a, sem_a)

        @pl.loop(0, hch, step=2)
        def _(i):
            wait_gather(i, rows_a, sem_a)
            start_gather(i + 1, rows_b, sem_b)
            process(i, rows_a)
            wait_gather(i + 1, rows_b, sem_b)

            @pl.when(i + 2 < hch)
            def _():
                start_gather(i + 2, rows_a, sem_a)

            process(i + 1, rows_b)

    plsc.subcore_barrier()

    @pl.loop(0, rpt // zrows)
    def _(k):
        r0 = sid * rpt + k * zrows
        pltpu.sync_copy(acc_sh.at[pl.ds(r0, zrows)], rows_a)
        pltpu.sync_copy(rows_a, out_hbm.at[cid, pl.ds(r0, zrows)])


def _edge_aggregate(y, src, dst, w):
    n, d = y.shape
    e = src.shape[0]
    hch = HCH
    totch = NS * (CPH0 + CPH1) * hch
    epad = totch * CH - e
    assert epad >= 0
    npad = NS * 640
    assert n <= npad
    if epad:
        src = jnp.concatenate([src, jnp.zeros((epad,), src.dtype)])
        pad_dst = n + jnp.arange(epad, dtype=dst.dtype) % (npad - n)
        dst = jnp.concatenate([dst, pad_dst])
        w = jnp.concatenate([w, jnp.zeros((epad,), w.dtype)])
    kern = functools.partial(
        pl.kernel,
        out_type=jax.ShapeDtypeStruct((NC, npad, d), _F32),
        mesh=_vector_mesh(),
        compiler_params=_sc_params(),
        scratch_types=[
            pltpu.VMEM_SHARED((npad, d), _F32),
            pltpu.VMEM((HCH, CH), jnp.int32),
            pltpu.VMEM((HCH, CH), jnp.int32),
            pltpu.VMEM((HCH, CH), _F32),
            pltpu.VMEM((CH, d), _F32),
            pltpu.VMEM((CH, d), _F32),
            pltpu.SemaphoreType.DMA,
            pltpu.SemaphoreType.DMA,
        ],
    )(_agg_body)
    return kern(y, src.reshape(totch, CH), dst.reshape(totch, CH),
                w.reshape(totch, CH))[:, :n]


def _leaky(v):
    return jnp.where(v >= 0, v, 0.01 * v)


def _k1_body(x_ref, w_ref, dinv_ref, y_ref):
    y_ref[...] = jnp.dot(x_ref[...], w_ref[...], precision=_HIGH,
                         preferred_element_type=_F32) * dinv_ref[...]


def _first_matmul(x, w1, dinv):
    return pl.pallas_call(
        _k1_body,
        out_shape=jax.ShapeDtypeStruct(x.shape, _F32),
    )(x, w1, dinv)


def _kmid_body(y_ref, parts_ref, dinv_ref, b_ref, w_ref, o_ref):
    h = y_ref[...] + parts_ref[0] + parts_ref[1]
    h = _leaky(dinv_ref[...] * h + b_ref[...])
    o_ref[...] = jnp.dot(h, w_ref[...], precision=_HIGH,
                         preferred_element_type=_F32) * dinv_ref[...]


def _mid_layer(y, parts, dinv, b, w_next):
    return pl.pallas_call(
        _kmid_body,
        out_shape=jax.ShapeDtypeStruct(y.shape, _F32),
    )(y, parts, dinv, b, w_next)


def _kfin_body(y_ref, parts_ref, dinv_ref, b_ref, batch_ref, wfc_ref,
               bfc_ref, o_ref):
    n = y_ref.shape[0]
    g = o_ref.shape[0]
    h = y_ref[...] + parts_ref[0] + parts_ref[1]
    h = dinv_ref[...] * h + b_ref[...]
    seg = lax.broadcasted_iota(jnp.int32, (n, g), 1)
    oh = (batch_ref[...] == seg).astype(_F32)
    sums = lax.dot_general(oh, h, (((0,), (0,)), ((), ())),
                           precision=_HIGH, preferred_element_type=_F32)
    counts = lax.dot_general(oh, jnp.ones((n, 1), _F32),
                             (((0,), (0,)), ((), ())),
                             precision=_HIGH, preferred_element_type=_F32)
    pooled = sums / jnp.maximum(counts, 1.0)
    logits = jnp.dot(pooled, wfc_ref[...], precision=_HIGH,
                     preferred_element_type=_F32) + bfc_ref[...]
    o_ref[...] = jax.nn.sigmoid(logits)


def _final_stage(y3, parts, dinv, b3, batch, wfc, bfc, g):
    o = wfc.shape[1]
    return pl.pallas_call(
        _kfin_body,
        out_shape=jax.ShapeDtypeStruct((g, o), _F32),
    )(y3, parts, dinv, b3, batch, wfc, bfc)


def kernel(x, edge_index, edge_weight, batch, W1, b1, W2, b2, W3, b3,
           Wfc, bfc):
    n, d = x.shape
    src = edge_index[0]
    dst = edge_index[1]

    degp = _deg_partials(dst, edge_weight, n)
    deg = jnp.sum(degp.reshape(NW, n), axis=0) + 1.0
    dinv = jnp.where(deg > 0, lax.rsqrt(deg), 0.0).reshape(n, 1)

    y1 = _first_matmul(x, W1, dinv)
    p1 = _edge_aggregate(y1, src, dst, edge_weight)
    y2 = _mid_layer(y1, p1, dinv, b1.reshape(1, d), W2)
    p2 = _edge_aggregate(y2, src, dst, edge_weight)
    y3 = _mid_layer(y2, p2, dinv, b2.reshape(1, d), W3)
    p3 = _edge_aggregate(y3, src, dst, edge_weight)

    return _final_stage(y3, p3, dinv, b3.reshape(1, d),
                        batch.reshape(n, 1), Wfc, bfc, g=64)


# speedup vs baseline: 1.4775x; 1.0003x over previous
import functools

import jax
import jax.numpy as jnp
from jax import lax
from jax.experimental import pallas as pl
from jax.experimental.pallas import tpu as pltpu
from jax.experimental.pallas import tpu_sc as plsc

NC = 2
NS = 16
NW = NC * NS
LANES = 16
CH = 128
CPH0 = 9
CPH1 = 1
HCH = 16

_F32 = jnp.float32
_HIGH = jax.lax.Precision.HIGHEST


def _vector_mesh():
    return plsc.VectorSubcoreMesh(core_axis_name="c", subcore_axis_name="s")


def _sc_params():
    import dataclasses
    cp = pltpu.CompilerParams()
    return dataclasses.replace(cp, needs_layout_passes=False)


def _deg_body(dst_hbm, w_hbm, out_hbm, acc, dstv, wv):
    n = acc.shape[1]
    ept = dst_hbm.shape[0] // NW
    cid = lax.axis_index("c")
    sid = lax.axis_index("s")
    wid = sid * NC + cid
    zeros16 = jnp.zeros((LANES,), _F32)
    zeros16i = jnp.zeros((LANES,), jnp.int32)

    @pl.loop(0, n, step=LANES)
    def _(j):
        acc[0, pl.ds(j, LANES)] = zeros16

    pltpu.sync_copy(dst_hbm.at[pl.ds(wid * ept, ept)], dstv)
    pltpu.sync_copy(w_hbm.at[pl.ds(wid * ept, ept)], wv)

    @pl.loop(0, ept, step=LANES)
    def _(j):
        plsc.addupdate_scatter(acc, [zeros16i, dstv[pl.ds(j, LANES)]],
                               wv[pl.ds(j, LANES)])

    pltpu.sync_copy(acc, out_hbm.at[wid])


def _deg_partials(dst, w, n):
    ept = dst.shape[0] // NW
    kern = functools.partial(
        pl.kernel,
        out_type=jax.ShapeDtypeStruct((NW, 1, n), _F32),
        mesh=_vector_mesh(),
        compiler_params=_sc_params(),
        scratch_types=[
            pltpu.VMEM((1, n), _F32),
            pltpu.VMEM((ept,), jnp.int32),
            pltpu.VMEM((ept,), _F32),
        ],
    )(_deg_body)
    return kern(dst, w)


def _agg_body(y_hbm, src_hbm, dst_hbm, w_hbm, out_hbm,
              acc_sh, src_v, dst_v, w_v, rows_a, rows_b,
              sem_a, sem_b):
    d = y_hbm.shape[1]
    npad = acc_sh.shape[0]
    hch = src_v.shape[0]
    rpt = npad // NS
    zrows = rows_a.shape[0]
    cid = lax.axis_index("c")
    sid = lax.axis_index("s")
    zeros16 = jnp.zeros((LANES,), _F32)

    nph = jnp.where(cid == 0, CPH0, CPH1)
    cb = jnp.where(cid == 0, sid * (CPH0 * hch),
                   NS * CPH0 * hch + sid * (CPH1 * hch))

    @pl.loop(0, zrows)
    def _(r):
        for cc in range(0, d, LANES):
            rows_a[r, pl.ds(cc, LANES)] = zeros16

    @pl.loop(0, rpt // zrows)
    def _(k):
        pltpu.sync_copy(rows_a,
                        acc_sh.at[pl.ds(sid * rpt + k * zrows, zrows)])

    plsc.subcore_barrier()

    def start_gather(j, buf, sem):
        pltpu.make_async_copy(y_hbm.at[src_v.at[j]], buf, sem).start()

    def wait_gather(j, buf, sem):
        pltpu.make_async_copy(y_hbm.at[src_v.at[j]], buf, sem).wait()

    def process(j, buf):
        wj = jnp.full((LANES,), j, jnp.int32)

        @pl.loop(0, CH, step=4)
        def _(r):
            for u in range(4):
                wsplat = plsc.load_gather(
                    w_v, [wj, jnp.full((LANES,), r + u, jnp.int32)])
                for cc in range(0, d, LANES):
                    buf[r + u, pl.ds(cc, LANES)] = (
                        buf[r + u, pl.ds(cc, LANES)] * wsplat)

        pltpu.sync_copy(buf, acc_sh.at[dst_v.at[j]], add=True)

    @pl.loop(0, nph)
    def _(ph):
        c0 = cb + ph * hch
        pltpu.sync_copy(src_hbm.at[pl.ds(c0, hch)], src_v)
        pltpu.sync_copy(dst_hbm.at[pl.ds(c0, hch)], dst_v)
        pltpu.sync_copy(w_hbm.at[pl.ds(c0, hch)], w_v)

        start_gather(0, rows_a, sem_a)

        @pl.loop(0, hch, step=2)
        def _(i):
            wait_gather(i, rows_a, sem_a)
            start_gather(i + 1, rows_b, sem_b)
            process(i, rows_a)
            wait_gather(i + 1, rows_b, sem_b)

            @pl.when(i + 2 < hch)
            def _():
                start_gather(i + 2, rows_a, sem_a)

            process(i + 1, rows_b)

    plsc.subcore_barrier()

    @pl.loop(0, rpt // zrows)
    def _(k):
        r0 = sid * rpt + k * zrows
        pltpu.sync_copy(acc_sh.at[pl.ds(r0, zrows)], rows_a)
        pltpu.sync_copy(rows_a, out_hbm.at[cid, pl.ds(r0, zrows)])


def _edge_aggregate(y, src, dst, w):
    n, d = y.shape
    e = src.shape[0]
    hch = HCH
    totch = NS * (CPH0 + CPH1) * hch
    epad = totch * CH - e
    assert epad >= 0
    npad = NS * 640
    assert n <= npad
    if epad:
        src = jnp.concatenate([src, jnp.zeros((epad,), src.dtype)])
        pad_dst = n + jnp.arange(epad, dtype=dst.dtype) % (npad - n)
        dst = jnp.concatenate([dst, pad_dst])
        w = jnp.concatenate([w, jnp.zeros((epad,), w.dtype)])
    kern = functools.partial(
        pl.kernel,
        out_type=jax.ShapeDtypeStruct((NC, npad, d), _F32),
        mesh=_vector_mesh(),
        compiler_params=_sc_params(),
        scratch_types=[
            pltpu.VMEM_SHARED((npad, d), _F32),
            pltpu.VMEM((HCH, CH), jnp.int32),
            pltpu.VMEM((HCH, CH), jnp.int32),
            pltpu.VMEM((HCH, CH), _F32),
            pltpu.VMEM((CH, d), _F32),
            pltpu.VMEM((CH, d), _F32),
            pltpu.SemaphoreType.DMA,
            pltpu.SemaphoreType.DMA,
        ],
    )(_agg_body)
    return kern(y, src.reshape(totch, CH), dst.reshape(totch, CH),
                w.reshape(totch, CH))[:, :n]


def _leaky(v):
    return jnp.where(v >= 0, v, 0.01 * v)


def _k1_body(x_ref, w_ref, dinv_ref, y_ref):
    y_ref[...] = jnp.dot(x_ref[...], w_ref[...], precision=_HIGH,
                         preferred_element_type=_F32) * dinv_ref[...]


def _first_matmul(x, w1, dinv):
    return pl.pallas_call(
        _k1_body,
        out_shape=jax.ShapeDtypeStruct(x.shape, _F32),
    )(x, w1, dinv)


def _kmid_body(y_ref, parts_ref, dinv_ref, b_ref, w_ref, o_ref):
    h = y_ref[...] + parts_ref[0] + parts_ref[1]
    h = _leaky(dinv_ref[...] * h + b_ref[...])
    o_ref[...] = jnp.dot(h, w_ref[...], precision=_HIGH,
                         preferred_element_type=_F32) * dinv_ref[...]


def _mid_layer(y, parts, dinv, b, w_next):
    return pl.pallas_call(
        _kmid_body,
        out_shape=jax.ShapeDtypeStruct(y.shape, _F32),
    )(y, parts, dinv, b, w_next)


def _kfin_body(y_ref, parts_ref, dinv_ref, b_ref, batch_ref, wfc_ref,
               bfc_ref, o_ref):
    n = y_ref.shape[0]
    g = o_ref.shape[0]
    h = y_ref[...] + parts_ref[0] + parts_ref[1]
    h = dinv_ref[...] * h + b_ref[...]
    seg = lax.broadcasted_iota(jnp.int32, (n, g), 1)
    oh = (batch_ref[...] == seg).astype(_F32)
    sums = lax.dot_general(oh, h, (((0,), (0,)), ((), ())),
                           precision=_HIGH, preferred_element_type=_F32)
    counts = lax.dot_general(oh, jnp.ones((n, 1), _F32),
                             (((0,), (0,)), ((), ())),
                             precision=_HIGH, preferred_element_type=_F32)
    pooled = sums / jnp.maximum(counts, 1.0)
    logits = jnp.dot(pooled, wfc_ref[...], precision=_HIGH,
                     preferred_element_type=_F32) + bfc_ref[...]
    o_ref[...] = jax.nn.sigmoid(logits)


def _final_stage(y3, parts, dinv, b3, batch, wfc, bfc, g):
    o = wfc.shape[1]
    return pl.pallas_call(
        _kfin_body,
        out_shape=jax.ShapeDtypeStruct((g, o), _F32),
    )(y3, parts, dinv, b3, batch, wfc, bfc)


def kernel(x, edge_index, edge_weight, batch, W1, b1, W2, b2, W3, b3,
           Wfc, bfc):
    n, d = x.shape
    src = edge_index[0]
    dst = edge_index[1]

    degp = _deg_partials(dst, edge_weight, n)
    deg = jnp.sum(degp.reshape(NW, n), axis=0) + 1.0
    dinv = jnp.where(deg > 0, lax.rsqrt(deg), 0.0).reshape(n, 1)

    y1 = _first_matmul(x, W1, dinv)
    p1 = _edge_aggregate(y1, src, dst, edge_weight)
    y2 = _mid_layer(y1, p1, dinv, b1.reshape(1, d), W2)
    p2 = _edge_aggregate(y2, src, dst, edge_weight)
    y3 = _mid_layer(y2, p2, dinv, b2.reshape(1, d), W3)
    p3 = _edge_aggregate(y3, src, dst, edge_weight)

    return _final_stage(y3, p3, dinv, b3.reshape(1, d),
                        batch.reshape(n, 1), Wfc, bfc, g=64)
